# baseline (speedup 1.0000x reference)
import functools

import jax
import jax.numpy as jnp
from jax import lax
from jax.experimental import pallas as pl
from jax.experimental.pallas import tpu as pltpu
from jax.experimental.pallas import tpu_sc as plsc

_DIM = 1152
_DIMP = 1280
_WPAD = _DIMP // 2
_WREAL = _DIM // 2
_GRP = _WREAL // 16
_BATCH = 16384
_ROWS_PAD = 1024
_NC = 2
_NS = 16
_NW = _NC * _NS
_BPW = _BATCH // _NW
_CHUNK = 32
_NCHUNK = _BPW // _CHUNK
_NBUF = 2


def _make_gather():
    mesh = plsc.VectorSubcoreMesh(core_axis_name="c", subcore_axis_name="s")

    @functools.partial(
        pl.kernel,
        mesh=mesh,
        out_type=jax.ShapeDtypeStruct((_BATCH, _DIM), jnp.float32),
        scratch_types=[
            pltpu.VMEM((_BPW,), jnp.int32),
            pltpu.VMEM((_CHUNK, _WPAD), jnp.int32),
            pltpu.VMEM((_CHUNK, _WPAD), jnp.int32),
            pltpu.VMEM((_CHUNK, _DIM), jnp.float32),
            pltpu.VMEM((_CHUNK, _DIM), jnp.float32),
            pltpu.SemaphoreType.DMA,
            pltpu.SemaphoreType.DMA,
            pltpu.SemaphoreType.DMA,
            pltpu.SemaphoreType.DMA,
        ],
    )
    def k(table_hbm, idx_hbm, out_hbm, idx_v, ib0, ib1, fb0, fb1,
          gs0, gs1, ws0, ws1):
        wid = lax.axis_index("s") * _NC + lax.axis_index("c")
        base = wid * _BPW
        pltpu.sync_copy(idx_hbm.at[pl.ds(base, _BPW)], idx_v)
        ibufs = (ib0, ib1)
        fbufs = (fb0, fb1)
        gsems = (gs0, gs1)
        wsems = (ws0, ws1)

        def gather_start(c):
            return pltpu.async_copy(
                table_hbm.at[idx_v.at[pl.ds(c * _CHUNK, _CHUNK)]],
                ibufs[c % _NBUF], gsems[c % _NBUF])

        def write_start(c):
            return pltpu.async_copy(
                fbufs[c % _NBUF], out_hbm.at[pl.ds(base + c * _CHUNK, _CHUNK)],
                wsems[c % _NBUF])

        def unpack(c):
            ib = ibufs[c % _NBUF]
            fb = fbufs[c % _NBUF]

            def row_body(r, carry):
                for w in range(_WPAD // 16):
                    x = ib[r, pl.ds(w * 16, 16)]
                    lo = lax.bitcast_convert_type(
                        lax.shift_left(x, jnp.int32(16)), jnp.float32)
                    fb[r, pl.ds(w * 16, 16)] = lo
                    if w < (_DIM - _WPAD) // 16:
                        hi = lax.bitcast_convert_type(
                            lax.bitwise_and(x, jnp.int32(-65536)),
                            jnp.float32)
                        fb[r, pl.ds(_WPAD + w * 16, 16)] = hi
                return carry

            lax.fori_loop(0, _CHUNK, row_body, 0)

        gcp = [None] * _NCHUNK
        wcp = [None] * _NCHUNK
        for c in range(_NBUF):
            gcp[c] = gather_start(c)
        for c in range(_NCHUNK):
            gcp[c].wait()
            if c >= _NBUF:
                wcp[c - _NBUF].wait()
            unpack(c)
            wcp[c] = write_start(c)
            if c + _NBUF < _NCHUNK:
                gcp[c + _NBUF] = gather_start(c + _NBUF)
        for c in range(_NCHUNK - _NBUF, _NCHUNK):
            wcp[c].wait()

    return k


_gather = _make_gather()


def kernel(labels, train, embedding_table):
    del train
    idx = labels.astype(jnp.int32)
    table_bf16 = jnp.concatenate(
        [embedding_table,
         jnp.zeros((_ROWS_PAD - embedding_table.shape[0], _DIM),
                   embedding_table.dtype)], axis=0).astype(jnp.bfloat16)
    lo = table_bf16[:, :_WPAD].view(jnp.uint16).astype(jnp.uint32)
    hi = jnp.concatenate(
        [table_bf16[:, _WPAD:],
         jnp.zeros((_ROWS_PAD, _DIMP - _DIM), jnp.bfloat16)],
        axis=1).view(jnp.uint16).astype(jnp.uint32)
    table_i32 = (lo | (hi << jnp.uint32(16))).view(jnp.int32)
    return _gather(table_i32, idx)


# speedup vs baseline: 1.6645x; 1.6645x over previous
import functools

import jax
import jax.numpy as jnp
from jax import lax
from jax.experimental import pallas as pl
from jax.experimental.pallas import tpu as pltpu
from jax.experimental.pallas import tpu_sc as plsc

_DIM = 1152
_BATCH = 16384
_NC = 2
_NS = 16
_NW = _NC * _NS
_BPW = _BATCH // _NW
_CHUNK = 32
_NCHUNK = _BPW // _CHUNK
_NBUF = 3


def _make_gather():
    mesh = plsc.VectorSubcoreMesh(core_axis_name="c", subcore_axis_name="s")

    @functools.partial(
        pl.kernel,
        mesh=mesh,
        out_type=jax.ShapeDtypeStruct((_BATCH, _DIM), jnp.float32),
        scratch_types=[
            pltpu.VMEM((_BPW,), jnp.int32),
            pltpu.VMEM((_CHUNK, _DIM), jnp.float32),
            pltpu.VMEM((_CHUNK, _DIM), jnp.float32),
            pltpu.VMEM((_CHUNK, _DIM), jnp.float32),
            pltpu.SemaphoreType.DMA,
            pltpu.SemaphoreType.DMA,
            pltpu.SemaphoreType.DMA,
            pltpu.SemaphoreType.DMA,
            pltpu.SemaphoreType.DMA,
            pltpu.SemaphoreType.DMA,
        ],
    )
    def k(table_hbm, idx_hbm, out_hbm, idx_v, buf0, buf1, buf2,
          gs0, gs1, gs2, ws0, ws1, ws2):
        wid = lax.axis_index("s") * _NC + lax.axis_index("c")
        base = wid * _BPW
        pltpu.sync_copy(idx_hbm.at[pl.ds(base, _BPW)], idx_v)
        bufs = (buf0, buf1, buf2)
        gsems = (gs0, gs1, gs2)
        wsems = (ws0, ws1, ws2)

        def gather_start(c):
            return pltpu.async_copy(
                table_hbm.at[idx_v.at[pl.ds(c * _CHUNK, _CHUNK)]],
                bufs[c % _NBUF], gsems[c % _NBUF])

        def write_start(c):
            return pltpu.async_copy(
                bufs[c % _NBUF], out_hbm.at[pl.ds(base + c * _CHUNK, _CHUNK)],
                wsems[c % _NBUF])

        gcp = [None] * _NCHUNK
        wcp = [None] * _NCHUNK
        for c in range(_NBUF):
            gcp[c] = gather_start(c)
        for c in range(_NCHUNK):
            gcp[c].wait()
            wcp[c] = write_start(c)
            if c + _NBUF < _NCHUNK:
                wcp[c].wait()
                gcp[c + _NBUF] = gather_start(c + _NBUF)
        for c in range(_NCHUNK - _NBUF, _NCHUNK):
            wcp[c].wait()

    return k


_gather = _make_gather()


def kernel(labels, train, embedding_table):
    del train
    return _gather(embedding_table, labels.astype(jnp.int32))
